# baseline (speedup 1.0000x reference)
import jax
import jax.numpy as jnp
from jax import lax
from jax.experimental import pallas as pl
from jax.experimental.pallas import tpu as pltpu
from jax.experimental.pallas import tpu_sc as plsc
from functools import partial

B = 32
C_IN = 96
H = 224
W_SP = 224
HW = H * W_SP
OUT_CH = 96
GROUP = 2
HIDDEN = OUT_CH // GROUP
K_ZERO = 23

B_SC = 8
B_TC = B - B_SC
N_WORKERS = 32
PLANES_SC = B_SC * C_IN
PER_W = PLANES_SC // N_WORKERS
LANE = 16


def _accum_plane(buf_ref):
    def row_body(i, acc):
        for j in range(W_SP // LANE):
            acc = acc + buf_ref[i, pl.ds(j * LANE, LANE)]
        return acc

    return lax.fori_loop(0, H, row_body, jnp.zeros((LANE,), jnp.float32))


def _sc_pool_body(x_hbm, out_hbm, buf_ref, res_ref, sem):
    wid = lax.axis_index("s") * 2 + lax.axis_index("c")
    for k in range(PER_W):
        p = wid * PER_W + k
        b = B_TC + p // C_IN
        c = p % C_IN
        pltpu.async_copy(x_hbm.at[b, c], buf_ref, sem).wait()
        res_ref[k] = _accum_plane(buf_ref)
    pltpu.sync_copy(res_ref, out_hbm.at[wid])


_sc_pool = partial(
    pl.kernel,
    out_type=jax.ShapeDtypeStruct((N_WORKERS, PER_W, LANE), jnp.float32),
    mesh=plsc.VectorSubcoreMesh(core_axis_name="c", subcore_axis_name="s"),
    scratch_types=[
        pltpu.VMEM((H, W_SP), jnp.float32),
        pltpu.VMEM((PER_W, LANE), jnp.float32),
        pltpu.SemaphoreType.DMA,
    ],
)(_sc_pool_body)


def _pool_sum_kernel(x_ref, out_ref):
    out_ref[0] = jnp.sum(x_ref[...], axis=(2, 3))


def _epilogue_kernel(tc_ref, sc_ref, wt_ref, b_ref, lasso_ref,
                     mask_ref, lasso_out_ref):
    sc_sums = jnp.sum(sc_ref[...], axis=2)
    sums = jnp.concatenate([tc_ref[...], sc_sums], axis=0)
    pooled = sums * (1.0 / HW)
    logits = jnp.dot(pooled, wt_ref[:, :],
                     preferred_element_type=jnp.float32) + b_ref[:, :]
    s = jax.nn.sigmoid(logits)
    lasso_out_ref[:, :] = lasso_ref[:, :] + jnp.sum(s) * (1.0 / B)
    s_i = s[:, :, None]
    s_j = s[:, None, :]
    i_idx = jax.lax.broadcasted_iota(jnp.int32, (B, HIDDEN, HIDDEN), 1)
    j_idx = jax.lax.broadcasted_iota(jnp.int32, (B, HIDDEN, HIDDEN), 2)
    lt = s_j < s_i
    eq_lo = (s_j == s_i) & (j_idx < i_idx)
    rank = jnp.sum((lt | eq_lo).astype(jnp.int32), axis=2)
    mask_ref[:, :] = ((rank >= K_ZERO) & (s > 0.0)).astype(jnp.int32)


def kernel(x, lasso_sum, W, b):
    sc_out = _sc_pool(x)
    sc_part = sc_out.reshape(B_SC, C_IN, LANE)

    tc_sums = pl.pallas_call(
        _pool_sum_kernel,
        grid=(B_TC,),
        in_specs=[pl.BlockSpec((1, C_IN, H, W_SP), lambda i: (i, 0, 0, 0))],
        out_specs=pl.BlockSpec((1, 1, C_IN), lambda i: (i, 0, 0)),
        out_shape=jax.ShapeDtypeStruct((B_TC, 1, C_IN), jnp.float32),
        compiler_params=pltpu.CompilerParams(
            dimension_semantics=("parallel",),
        ),
    )(x)

    wt = W.T
    b2 = b.reshape(1, HIDDEN)
    lasso2 = lasso_sum.reshape(1, 1)

    mask, lasso_out = pl.pallas_call(
        _epilogue_kernel,
        in_specs=[
            pl.BlockSpec((B_TC, C_IN), lambda: (0, 0)),
            pl.BlockSpec((B_SC, C_IN, LANE), lambda: (0, 0, 0)),
            pl.BlockSpec((C_IN, HIDDEN), lambda: (0, 0)),
            pl.BlockSpec((1, HIDDEN), lambda: (0, 0)),
            pl.BlockSpec((1, 1), lambda: (0, 0)),
        ],
        out_specs=[
            pl.BlockSpec((B, HIDDEN), lambda: (0, 0)),
            pl.BlockSpec((1, 1), lambda: (0, 0)),
        ],
        out_shape=[
            jax.ShapeDtypeStruct((B, HIDDEN), jnp.int32),
            jax.ShapeDtypeStruct((1, 1), jnp.float32),
        ],
    )(tc_sums.reshape(B_TC, C_IN), sc_part, wt, b2, lasso2)

    expanded = jnp.reshape(
        jnp.broadcast_to(mask[:, :, None], (B, HIDDEN, GROUP)), (B, OUT_CH)
    )
    return expanded, lasso_out.reshape(())
